# baseline (speedup 1.0000x reference)
import functools

import jax
import jax.numpy as jnp
from jax import lax
from jax.experimental import pallas as pl
from jax.experimental.pallas import tpu as pltpu
from jax.experimental.pallas import tpu_sc as plsc

LATENT = 128
NCLASS = 1000
NCLASS_PAD = 1024
BATCH = 16384
SC_ROWS = 8192
TC_ROWS = BATCH - SC_ROWS
NC, NS, L = 2, 16, 16
NW = NC * NS
BPW = SC_ROWS // NW
CH = 128
NCHUNK = BPW // CH
TCB = 256

_mesh = plsc.VectorSubcoreMesh(core_axis_name="c", subcore_axis_name="s")


@functools.partial(
    pl.kernel,
    mesh=_mesh,
    out_type=jax.ShapeDtypeStruct((SC_ROWS, LATENT), jnp.float32),
    scratch_types=[
        pltpu.VMEM((BPW,), jnp.int32),
        pltpu.VMEM((BPW, LATENT), jnp.float32),
        pltpu.VMEM((CH, LATENT), jnp.float32),
        pltpu.VMEM((CH, LATENT), jnp.float32),
        pltpu.SemaphoreType.DMA,
        pltpu.SemaphoreType.DMA,
        pltpu.SemaphoreType.DMA,
        pltpu.SemaphoreType.DMA,
        pltpu.SemaphoreType.DMA,
        pltpu.SemaphoreType.DMA,
    ],
)
def _emb_mul_sc(z_hbm, label_hbm, table_hbm, out_hbm, idx_v, zb, r0, r1,
                sg0, sg1, sz0, sz1, so0, so1):
    wid = lax.axis_index("s") * NC + lax.axis_index("c")
    base = wid * BPW
    rbuf = (r0, r1)
    sg = (sg0, sg1)
    sz = (sz0, sz1)
    so = (so0, so1)

    pltpu.sync_copy(label_hbm.at[pl.ds(base, BPW)], idx_v)
    z_cp = [None] * NCHUNK
    g_cp = [None] * NCHUNK
    for c in range(NCHUNK):
        g_cp[c] = pltpu.async_copy(
            table_hbm.at[idx_v.at[pl.ds(c * CH, CH)]], rbuf[c], sg[c])
        z_cp[c] = pltpu.async_copy(
            z_hbm.at[pl.ds(base + c * CH, CH)],
            zb.at[pl.ds(c * CH, CH)], sz[c])

    out_cp = [None] * NCHUNK
    for c in range(NCHUNK):
        g_cp[c].wait()
        z_cp[c].wait()
        rb = rbuf[c]

        @plsc.parallel_loop(0, CH, step=1, unroll=2)
        def row(r):
            zr = c * CH + r
            for j in range(LATENT // L):
                s = pl.ds(j * L, L)
                zb[zr, s] = zb[zr, s] * rb[r, s]

        out_cp[c] = pltpu.async_copy(
            zb.at[pl.ds(c * CH, CH)],
            out_hbm.at[pl.ds(base + c * CH, CH)], so[c])
    for c in range(NCHUNK):
        out_cp[c].wait()


def _emb_mul_tc_body(lab_ref, z_ref, tab_ref, out_ref):
    lab = lab_ref[0]
    kcol = lax.broadcasted_iota(jnp.int32, (NCLASS_PAD, 1), 0)
    oh = (kcol == lab).astype(jnp.float32)
    rows = lax.dot_general(oh, tab_ref[...],
                           (((0,), (0,)), ((), ())),
                           preferred_element_type=jnp.float32)
    out_ref[...] = z_ref[...] * rows


_emb_mul_tc = pl.pallas_call(
    _emb_mul_tc_body,
    grid=(TC_ROWS // TCB,),
    in_specs=[
        pl.BlockSpec((1, 1, TCB), lambda i: (i, 0, 0)),
        pl.BlockSpec((TCB, LATENT), lambda i: (i, 0)),
        pl.BlockSpec((NCLASS_PAD, LATENT), lambda i: (0, 0)),
    ],
    out_specs=pl.BlockSpec((TCB, LATENT), lambda i: (i, 0)),
    out_shape=jax.ShapeDtypeStruct((TC_ROWS, LATENT), jnp.float32),
)


def kernel(z, label, table):
    label = label.astype(jnp.int32)
    tab_pad = jnp.zeros((NCLASS_PAD, LATENT), jnp.float32).at[:NCLASS].set(table)
    sc_out = _emb_mul_sc(z[:SC_ROWS], label[:SC_ROWS], table)
    lab_tc = label[SC_ROWS:].reshape(TC_ROWS // TCB, 1, TCB)
    tc_out = _emb_mul_tc(lab_tc, z[SC_ROWS:], tab_pad)
    return jnp.concatenate([sc_out, tc_out], axis=0)


# speedup vs baseline: 1.7728x; 1.7728x over previous
import functools

import jax
import jax.numpy as jnp
from jax import lax
from jax.experimental import pallas as pl
from jax.experimental.pallas import tpu as pltpu
from jax.experimental.pallas import tpu_sc as plsc

LATENT = 128
BATCH = 16384
NC, NS, L = 2, 16, 16
NW = NC * NS
BPW = BATCH // NW
CH = 128
NCHUNK = BPW // CH
NRB = 3

_mesh = plsc.VectorSubcoreMesh(core_axis_name="c", subcore_axis_name="s")


@functools.partial(
    pl.kernel,
    mesh=_mesh,
    out_type=jax.ShapeDtypeStruct((BATCH, LATENT), jnp.float32),
    scratch_types=[
        pltpu.VMEM((BPW,), jnp.int32),
        pltpu.VMEM((BPW, LATENT), jnp.float32),
        pltpu.VMEM((CH, LATENT), jnp.float32),
        pltpu.VMEM((CH, LATENT), jnp.float32),
        pltpu.VMEM((CH, LATENT), jnp.float32),
        pltpu.SemaphoreType.DMA,
        pltpu.SemaphoreType.DMA,
        pltpu.SemaphoreType.DMA,
        pltpu.SemaphoreType.DMA,
        pltpu.SemaphoreType.DMA,
        pltpu.SemaphoreType.DMA,
        pltpu.SemaphoreType.DMA,
        pltpu.SemaphoreType.DMA,
        pltpu.SemaphoreType.DMA,
    ],
)
def _emb_mul(z_hbm, label_hbm, table_hbm, out_hbm, idx_v, zb, r0, r1, r2,
             sg0, sg1, sg2, sz0, sz1, sz2, sz3, so0, so1):
    wid = lax.axis_index("s") * NC + lax.axis_index("c")
    base = wid * BPW
    rbuf = (r0, r1, r2)
    sg = (sg0, sg1, sg2)
    sz = (sz0, sz1, sz2, sz3)
    so = (so0, so1)

    pltpu.sync_copy(label_hbm.at[pl.ds(base, BPW)], idx_v)
    z_cp = [None] * NCHUNK
    g_cp = [None] * NCHUNK
    for c in range(NRB):
        g_cp[c] = pltpu.async_copy(
            table_hbm.at[idx_v.at[pl.ds(c * CH, CH)]], rbuf[c], sg[c])
        z_cp[c] = pltpu.async_copy(
            z_hbm.at[pl.ds(base + c * CH, CH)],
            zb.at[pl.ds(c * CH, CH)], sz[c])
    z_cp[NCHUNK - 1] = pltpu.async_copy(
        z_hbm.at[pl.ds(base + (NCHUNK - 1) * CH, CH)],
        zb.at[pl.ds((NCHUNK - 1) * CH, CH)], sz[NCHUNK - 1])

    out_cp = [None] * NCHUNK
    for c in range(NCHUNK):
        b = c % NRB
        g_cp[c].wait()
        z_cp[c].wait()
        rb = rbuf[b]

        @plsc.parallel_loop(0, CH, step=1, unroll=2)
        def row(r):
            zr = c * CH + r
            for j in range(LATENT // L):
                s = pl.ds(j * L, L)
                zb[zr, s] = zb[zr, s] * rb[r, s]

        if c + NRB < NCHUNK:
            g_cp[c + NRB] = pltpu.async_copy(
                table_hbm.at[idx_v.at[pl.ds((c + NRB) * CH, CH)]],
                rbuf[b], sg[b])
        out_cp[c] = pltpu.async_copy(
            zb.at[pl.ds(c * CH, CH)],
            out_hbm.at[pl.ds(base + c * CH, CH)], so[c % 2])
    for c in range(NCHUNK):
        out_cp[c].wait()


def kernel(z, label, table):
    return _emb_mul(z, label.astype(jnp.int32), table)


# speedup vs baseline: 1.7828x; 1.0057x over previous
import functools

import jax
import jax.numpy as jnp
from jax import lax
from jax.experimental import pallas as pl
from jax.experimental.pallas import tpu as pltpu
from jax.experimental.pallas import tpu_sc as plsc

LATENT = 128
BATCH = 16384
NC, NS, L = 2, 16, 16
NW = NC * NS
BPW = BATCH // NW
CH = 128
NCHUNK = BPW // CH
NRB = 3

_mesh = plsc.VectorSubcoreMesh(core_axis_name="c", subcore_axis_name="s")


@functools.partial(
    pl.kernel,
    mesh=_mesh,
    out_type=jax.ShapeDtypeStruct((BATCH, LATENT), jnp.float32),
    scratch_types=[
        pltpu.VMEM((BPW,), jnp.int32),
        pltpu.VMEM((BPW, LATENT), jnp.float32),
        pltpu.VMEM((CH, LATENT), jnp.float32),
        pltpu.VMEM((CH, LATENT), jnp.float32),
        pltpu.VMEM((CH, LATENT), jnp.float32),
        pltpu.SemaphoreType.DMA,
        pltpu.SemaphoreType.DMA,
        pltpu.SemaphoreType.DMA,
        pltpu.SemaphoreType.DMA,
        pltpu.SemaphoreType.DMA,
        pltpu.SemaphoreType.DMA,
        pltpu.SemaphoreType.DMA,
        pltpu.SemaphoreType.DMA,
        pltpu.SemaphoreType.DMA,
    ],
)
def _emb_mul(z_hbm, label_hbm, table_hbm, out_hbm, idx_v, zb, r0, r1, r2,
             sg0, sg1, sg2, sz0, sz1, sz2, sz3, so0, so1):
    wid = lax.axis_index("s") * NC + lax.axis_index("c")
    base = wid * BPW
    rbuf = (r0, r1, r2)
    sg = (sg0, sg1, sg2)
    sz = (sz0, sz1, sz2, sz3)
    so = (so0, so1)

    pltpu.sync_copy(label_hbm.at[pl.ds(base, BPW)], idx_v)
    z_cp = [None] * NCHUNK
    g_cp = [None] * NCHUNK
    for c in range(NRB):
        g_cp[c] = pltpu.async_copy(
            table_hbm.at[idx_v.at[pl.ds(c * CH, CH)]], rbuf[c], sg[c])
        z_cp[c] = pltpu.async_copy(
            z_hbm.at[pl.ds(base + c * CH, CH)],
            zb.at[pl.ds(c * CH, CH)], sz[c])
    z_cp[NCHUNK - 1] = pltpu.async_copy(
        z_hbm.at[pl.ds(base + (NCHUNK - 1) * CH, CH)],
        zb.at[pl.ds((NCHUNK - 1) * CH, CH)], sz[NCHUNK - 1])

    out_cp = [None] * NCHUNK
    for c in range(NCHUNK):
        b = c % NRB
        g_cp[c].wait()
        z_cp[c].wait()
        rb = rbuf[b]

        @plsc.parallel_loop(0, CH, step=1, unroll=2)
        def row(r):
            zr = c * CH + r
            for j in range(LATENT // L):
                s = pl.ds(j * L, L)
                zb[zr, s] = zb[zr, s] * rb[r, s]

        out_cp[c] = pltpu.async_copy(
            zb.at[pl.ds(c * CH, CH)],
            out_hbm.at[pl.ds(base + c * CH, CH)], so[c % 2])
        if c + NRB < NCHUNK:
            g_cp[c + NRB] = pltpu.async_copy(
                table_hbm.at[idx_v.at[pl.ds((c + NRB) * CH, CH)]],
                rbuf[b], sg[b])
    for c in range(NCHUNK):
        out_cp[c].wait()


def kernel(z, label, table):
    return _emb_mul(z, label.astype(jnp.int32), table)
